# baseline (speedup 1.0000x reference)
import jax
import jax.numpy as jnp
from jax import lax
from jax.experimental import pallas as pl
from jax.experimental.pallas import tpu as pltpu
from jax.experimental.pallas import tpu_sc as plsc

BATCH = 16384
DIM = 128
NSITES = 16
LANES = 16
NC, NS = 2, 16
NW = NC * NS
BPW = BATCH // NW
CH = 128
NCHUNK = BPW // CH
NBUF = 2


def _body(in_hbm, sites_hbm, masks_hbm, idx_hbm, out_hbm,
          idx_v, sites_v, masks_v, in_buf, out_buf,
          sem_in, sem_out):
    wid = lax.axis_index("s") * NC + lax.axis_index("c")
    base = wid * BPW

    pltpu.sync_copy(idx_hbm.at[pl.ds(base, BPW)], idx_v)
    pltpu.sync_copy(sites_hbm, sites_v)
    pltpu.sync_copy(masks_hbm, masks_v)

    in_copies = [None] * NCHUNK
    out_copies = [None] * NCHUNK
    in_copies[0] = pltpu.async_copy(
        in_hbm.at[pl.ds(base, CH)], in_buf.at[0], sem_in.at[0])

    for g in range(NCHUNK):
        slot = g % NBUF
        if g + 1 < NCHUNK:
            nslot = (g + 1) % NBUF
            in_copies[g + 1] = pltpu.async_copy(
                in_hbm.at[pl.ds(base + (g + 1) * CH, CH)],
                in_buf.at[nslot], sem_in.at[nslot])
        in_copies[g].wait()
        if g >= NBUF:
            out_copies[g - NBUF].wait()

        zeros16 = jnp.zeros((LANES,), dtype=jnp.int32)

        def row(r, carry, slot=slot, off=g * CH):
            jvec = plsc.load_gather(idx_v, [zeros16 + (off + r)])
            mi = plsc.load_gather(masks_v, [jvec])
            si = plsc.load_gather(sites_v, [jvec])
            for q in range(DIM // LANES):
                v = in_buf[slot, r, pl.ds(q * LANES, LANES)]
                out_buf[slot, r, pl.ds(q * LANES, LANES)] = v * mi + si
            return carry

        lax.fori_loop(0, CH, row, 0)

        out_copies[g] = pltpu.async_copy(
            out_buf.at[slot], out_hbm.at[pl.ds(base + g * CH, CH)],
            sem_out.at[slot])

    for g in range(max(0, NCHUNK - NBUF), NCHUNK):
        out_copies[g].wait()


def kernel(inputs, injection_sites, masks, random_indexes):
    mesh = plsc.VectorSubcoreMesh(core_axis_name="c", subcore_axis_name="s")
    k = pl.kernel(
        _body,
        out_type=jax.ShapeDtypeStruct((BATCH, DIM), jnp.float32),
        mesh=mesh,
        compiler_params=pltpu.CompilerParams(needs_layout_passes=False),
        scratch_types=[
            pltpu.VMEM((BPW,), jnp.int32),
            pltpu.VMEM((NSITES,), jnp.float32),
            pltpu.VMEM((NSITES,), jnp.float32),
            pltpu.VMEM((NBUF, CH, DIM), jnp.float32),
            pltpu.VMEM((NBUF, CH, DIM), jnp.float32),
            pltpu.SemaphoreType.DMA((NBUF,)),
            pltpu.SemaphoreType.DMA((NBUF,)),
        ],
    )
    return k(inputs,
             injection_sites.reshape(NSITES).astype(jnp.float32),
             masks.reshape(NSITES).astype(jnp.float32),
             random_indexes.astype(jnp.int32))


# speedup vs baseline: 1.3855x; 1.3855x over previous
import jax
import jax.numpy as jnp
from jax import lax
from jax.experimental import pallas as pl
from jax.experimental.pallas import tpu as pltpu
from jax.experimental.pallas import tpu_sc as plsc

BATCH = 16384
DIM = 128
NSITES = 16
LANES = 16
NC, NS = 2, 16
NW = NC * NS
BPW = BATCH // NW
CH = 128
NCHUNK = BPW // CH
NBUF = 2


def _body(in_hbm, sites_hbm, masks_hbm, idx_hbm, out_hbm,
          idx_v, sites_v, masks_v, in_buf, out_buf,
          sem_in, sem_out):
    wid = lax.axis_index("s") * NC + lax.axis_index("c")
    base = wid * BPW

    pltpu.sync_copy(idx_hbm.at[pl.ds(base, BPW)], idx_v)
    pltpu.sync_copy(sites_hbm, sites_v)
    pltpu.sync_copy(masks_hbm, masks_v)

    in_copies = [None] * NCHUNK
    out_copies = [None] * NCHUNK
    in_copies[0] = pltpu.async_copy(
        in_hbm.at[pl.ds(base, CH)], in_buf.at[0], sem_in.at[0])

    for g in range(NCHUNK):
        slot = g % NBUF
        if g + 1 < NCHUNK:
            nslot = (g + 1) % NBUF
            in_copies[g + 1] = pltpu.async_copy(
                in_hbm.at[pl.ds(base + (g + 1) * CH, CH)],
                in_buf.at[nslot], sem_in.at[nslot])
        in_copies[g].wait()
        if g >= NBUF:
            out_copies[g - NBUF].wait()

        zeros16 = jnp.zeros((LANES,), dtype=jnp.int32)

        @plsc.parallel_loop(0, CH, unroll=4)
        def _rows(r, slot=slot, off=g * CH):
            jvec = plsc.load_gather(idx_v, [zeros16 + (off + r)])
            mi = plsc.load_gather(masks_v, [jvec])
            si = plsc.load_gather(sites_v, [jvec])
            for q in range(DIM // LANES):
                v = in_buf[slot, r, pl.ds(q * LANES, LANES)]
                out_buf[slot, r, pl.ds(q * LANES, LANES)] = v * mi + si

        out_copies[g] = pltpu.async_copy(
            out_buf.at[slot], out_hbm.at[pl.ds(base + g * CH, CH)],
            sem_out.at[slot])

    for g in range(max(0, NCHUNK - NBUF), NCHUNK):
        out_copies[g].wait()


def kernel(inputs, injection_sites, masks, random_indexes):
    mesh = plsc.VectorSubcoreMesh(core_axis_name="c", subcore_axis_name="s")
    k = pl.kernel(
        _body,
        out_type=jax.ShapeDtypeStruct((BATCH, DIM), jnp.float32),
        mesh=mesh,
        compiler_params=pltpu.CompilerParams(needs_layout_passes=False),
        scratch_types=[
            pltpu.VMEM((BPW,), jnp.int32),
            pltpu.VMEM((NSITES,), jnp.float32),
            pltpu.VMEM((NSITES,), jnp.float32),
            pltpu.VMEM((NBUF, CH, DIM), jnp.float32),
            pltpu.VMEM((NBUF, CH, DIM), jnp.float32),
            pltpu.SemaphoreType.DMA((NBUF,)),
            pltpu.SemaphoreType.DMA((NBUF,)),
        ],
    )
    return k(inputs,
             injection_sites.reshape(NSITES).astype(jnp.float32),
             masks.reshape(NSITES).astype(jnp.float32),
             random_indexes.astype(jnp.int32))
